# baseline (speedup 1.0000x reference)
import functools

import jax
import jax.numpy as jnp
from jax import lax
from jax.experimental import pallas as pl
from jax.experimental.pallas import tpu as pltpu
from jax.experimental.pallas import tpu_sc as plsc

_B = 4096
_V = 100000
_L = 16

_NS = 16
_BPW = _B // _NS
_NCH = _BPW // 128
_GROUPS = 128 // _L

_ITILES = _B // 128


def _sc_gather_mean(flat, target):
  mesh = plsc.VectorSubcoreMesh(
      core_axis_name="c", subcore_axis_name="s", num_cores=1)

  @functools.partial(
      pl.kernel,
      mesh=mesh,
      compiler_params=pltpu.CompilerParams(needs_layout_passes=False),
      out_type=(
          jax.ShapeDtypeStruct((_NS, _L), jnp.float32),
          jax.ShapeDtypeStruct((_L,), jnp.float32),
      ),
      scratch_types=[
          pltpu.VMEM((_BPW,), jnp.int32),
          pltpu.VMEM((_NCH, 128), jnp.int32),
          pltpu.VMEM((_NCH, 128), jnp.float32),
          pltpu.VMEM((_L,), jnp.float32),
          pltpu.VMEM((_NS, _L), jnp.float32),
          pltpu.SemaphoreType.DMA,
      ],
  )
  def sc_kernel(flat_hbm, tgt_hbm, stage_hbm, out_hbm,
                tgt_v, idx_v, vals_v, acc_v, all_v, sem):
    sid = lax.axis_index("s")
    base = sid * _BPW

    pltpu.sync_copy(tgt_hbm.at[pl.ds(base, _BPW)], tgt_v)

    lane_iota = lax.iota(jnp.int32, _L)

    def idx_body(g, _):
      o = g * _L
      j = tgt_v[pl.ds(o, _L)]
      i = (base + o) + lane_iota
      blk = lax.shift_right_logical(j, 3) * _ITILES + lax.shift_right_logical(i, 7)
      sub = lax.shift_left(lax.bitwise_and(j, 7), 7) + lax.bitwise_and(i, 127)
      idx_v[lax.div(g, _GROUPS), pl.ds(lax.rem(g, _GROUPS) * _L, _L)] = (
          lax.shift_left(blk, 10) + sub)
      return 0

    lax.fori_loop(0, _NCH * _GROUPS, idx_body, 0)

    copies = [
        pltpu.async_copy(flat_hbm.at[idx_v.at[k]], vals_v.at[k], sem)
        for k in range(_NCH)
    ]
    for c in copies:
      c.wait()

    def sum_body(g, acc):
      return acc + vals_v[lax.div(g, _GROUPS), pl.ds(lax.rem(g, _GROUPS) * _L, _L)]

    acc = lax.fori_loop(0, _NCH * _GROUPS, sum_body, jnp.zeros((_L,), jnp.float32))
    acc_v[...] = acc

    pltpu.sync_copy(acc_v, stage_hbm.at[sid])
    plsc.subcore_barrier()

    @pl.when(sid == 0)
    def _():
      pltpu.sync_copy(stage_hbm, all_v)
      tot = jnp.zeros((_L,), jnp.float32)
      for r in range(_NS):
        tot = tot + all_v[r]
      s = lax.reduce_sum(tot, axes=(0,))
      acc_v[...] = jax.lax.broadcast(s * (-1.0 / _B), (_L,))
      pltpu.sync_copy(acc_v, out_hbm)

  return sc_kernel(flat, target)


def kernel(input, target):
  flat = (
      input.T.reshape(_V // 8, 8, _ITILES, 128)
      .transpose(0, 2, 1, 3)
      .reshape(_B * _V)
  )
  tgt = target.astype(jnp.int32)
  _, final = _sc_gather_mean(flat, tgt)
  return final[0]
